# baseline (speedup 1.0000x reference)
import functools

import jax
import jax.numpy as jnp
from jax import lax
from jax.experimental import pallas as pl
from jax.experimental.pallas import tpu as pltpu
from jax.experimental.pallas import tpu_sc as plsc

EPS = 1e-3


def _make_sc_gather(V, D, N, CH=256):
    info = plsc.get_sparse_core_info()
    NC, NS = info.num_cores, info.num_subcores
    NW = NC * NS
    n_per_w = N // NW
    n_chunks = n_per_w // CH
    assert n_per_w % CH == 0 and n_chunks % 2 == 0

    mesh = plsc.VectorSubcoreMesh(core_axis_name="c", subcore_axis_name="s")

    @functools.partial(
        pl.kernel,
        mesh=mesh,
        out_type=jax.ShapeDtypeStruct((N, D), jnp.float32),
        scratch_types=[
            pltpu.VMEM((n_per_w,), jnp.int32),
            pltpu.VMEM((CH, D), jnp.float32),
            pltpu.VMEM((CH, D), jnp.float32),
            pltpu.SemaphoreType.DMA,
            pltpu.SemaphoreType.DMA,
        ],
    )
    def gather_k(idx_hbm, table_hbm, out_hbm, idx_v, rows0, rows1, sem0, sem1):
        wid = lax.axis_index("s") * NC + lax.axis_index("c")
        base = wid * n_per_w
        pltpu.sync_copy(idx_hbm.at[pl.ds(base, n_per_w)], idx_v)

        def g_start(i, rows, sem):
            return pltpu.async_copy(
                table_hbm.at[idx_v.at[pl.ds(i * CH, CH)]], rows, sem)

        def g_wait(i, rows, sem):
            pltpu.make_async_copy(
                table_hbm.at[idx_v.at[pl.ds(i * CH, CH)]], rows, sem).wait()

        def put(i, rows):
            pltpu.sync_copy(rows, out_hbm.at[pl.ds(base + i * CH, CH)])

        g_start(0, rows0, sem0)

        def body(j, carry):
            i = 2 * j
            g_start(i + 1, rows1, sem1)
            g_wait(i, rows0, sem0)
            put(i, rows0)
            g_start(i + 2, rows0, sem0)
            g_wait(i + 1, rows1, sem1)
            put(i + 1, rows1)
            return carry

        lax.fori_loop(0, n_chunks // 2 - 1, body, 0)
        g_start(n_chunks - 1, rows1, sem1)
        g_wait(n_chunks - 2, rows0, sem0)
        put(n_chunks - 2, rows0)
        g_wait(n_chunks - 1, rows1, sem1)
        put(n_chunks - 1, rows1)

    return gather_k


def _ln_body(sum_ref, pos_ref, tt_ref, type_ref, gamma_ref, beta_ref, out_ref):
    x = sum_ref[...] + pos_ref[...]
    ttf = tt_ref[...]
    t0 = type_ref[0:1, :]
    t1 = type_ref[1:2, :]
    x = x + t0 + ttf * (t1 - t0)
    mean = jnp.mean(x, axis=-1, keepdims=True)
    xc = x - mean
    var = jnp.mean(xc * xc, axis=-1, keepdims=True)
    y = xc * lax.rsqrt(var + EPS)
    out_ref[...] = y * gamma_ref[...] + beta_ref[...]


def _ln_call(summed, position_table, tt3, type_table, gamma2, beta2, B, S, D):
    return pl.pallas_call(
        _ln_body,
        grid=(B,),
        in_specs=[
            pl.BlockSpec((S, D), lambda i: (i, 0)),
            pl.BlockSpec((S, D), lambda i: (0, 0)),
            pl.BlockSpec((S, 1), lambda i: (i, 0)),
            pl.BlockSpec((2, D), lambda i: (0, 0)),
            pl.BlockSpec((1, D), lambda i: (0, 0)),
            pl.BlockSpec((1, D), lambda i: (0, 0)),
        ],
        out_specs=pl.BlockSpec((S, D), lambda i: (i, 0)),
        out_shape=jax.ShapeDtypeStruct((B * S, D), jnp.float32),
    )(summed, position_table, tt3, type_table, gamma2, beta2)


def kernel(input_ids, token_type_ids, token_embedding, position_table, type_table, gamma, beta):
    B, S = input_ids.shape
    V, D = token_embedding.shape
    N = B * S

    idx_flat = input_ids.reshape(N).astype(jnp.int32)
    gathered = _make_sc_gather(V, D, N)(idx_flat, token_embedding)

    ttf = token_type_ids.reshape(N, 1).astype(jnp.float32)
    out = _ln_call(
        gathered, position_table, ttf, type_table,
        gamma.reshape(1, D), beta.reshape(1, D), B, S, D,
    )
    return out.reshape(B, S, D), token_embedding
